# baseline (speedup 1.0000x reference)
import functools

import jax
import jax.numpy as jnp
import numpy as np
from jax import lax
from jax.experimental import pallas as pl
from jax.experimental.pallas import tpu as pltpu
from jax.experimental.pallas import tpu_sc as plsc

N_NODES = 10000
N_EDGES = 320000
D_IN = 128
D_OUT = 16
N_HEAD = 8
D_FLAT = D_OUT * N_HEAD
D_MSG = (D_OUT + 1) * N_HEAD
D_PAD = 144

NC, NS = 2, 16
NW = NC * NS
EPW = N_EDGES // NW
CHUNK = 128
NCHUNK = EPW // CHUNK
TAIL = EPW - NCHUNK * CHUNK
ROWS_PER_TILE = N_NODES // NS

_P = np.tile(np.eye(N_HEAD, dtype=np.float32), (D_OUT, D_OUT))

_Q_np = np.zeros((D_PAD, D_FLAT), dtype=np.float32)
for _j in range(N_HEAD):
    for _i in range(D_OUT):
        _Q_np[D_FLAT + _j, _i * N_HEAD + _j] = 1.0
_Q = _Q_np


def _stage1_body(x_ref, w_ref, a2_ref, p_ref, out_ref):
    s0 = jnp.dot(x_ref[...], w_ref[...], preferred_element_type=jnp.float32)
    t = s0 * a2_ref[...]
    attn2b = jnp.dot(t, p_ref[...], preferred_element_type=jnp.float32)
    z = attn2b + jnp.sqrt(attn2b * attn2b + 1.0)
    out_ref[...] = jnp.concatenate(
        [s0 * z, z[:, :N_HEAD], jnp.zeros_like(z[:, :N_HEAD])], axis=1)


def _stage1(x, W, a2f):
    blk = 1000
    grid = N_NODES // blk
    return pl.pallas_call(
        _stage1_body,
        grid=(grid,),
        in_specs=[
            pl.BlockSpec((blk, D_IN), lambda i: (i, 0)),
            pl.BlockSpec((D_IN, D_FLAT), lambda i: (0, 0)),
            pl.BlockSpec((1, D_FLAT), lambda i: (0, 0)),
            pl.BlockSpec((D_FLAT, D_FLAT), lambda i: (0, 0)),
        ],
        out_specs=pl.BlockSpec((blk, D_PAD), lambda i: (i, 0)),
        out_shape=jax.ShapeDtypeStruct((N_NODES, D_PAD), jnp.float32),
    )(x, W, a2f, _P)


def _spmm_body(eidx_hbm, adj_hbm, feat_hbm, out_hbm,
               rbuf, cbuf, abuf, sbuf, rows, rt, ct, at_, rowt, acc,
               semi0, semi1, semg0, semg1, sems0, sems1):
    c = lax.axis_index("c")
    s = lax.axis_index("s")
    wid = s * NC + c

    isems = (semi0, semi1)
    gsems = (semg0, semg1)
    ssems = (sems0, sems1)

    ebase = wid * EPW

    def _start_idx(ci, b):
        off = pl.multiple_of(ebase + ci * CHUNK, 8)
        pltpu.async_copy(eidx_hbm.at[0, pl.ds(off, CHUNK)],
                         rbuf.at[b], isems[b])
        pltpu.async_copy(eidx_hbm.at[1, pl.ds(off, CHUNK)],
                         cbuf.at[b], isems[b])
        pltpu.async_copy(adj_hbm.at[pl.ds(off, CHUNK)],
                         abuf.at[b], isems[b])

    def _wait_idx(b):
        src = eidx_hbm.at[0, pl.ds(0, CHUNK)]
        pltpu.make_async_copy(src, rbuf.at[b], isems[b]).wait()
        pltpu.make_async_copy(src, cbuf.at[b], isems[b]).wait()
        pltpu.make_async_copy(adj_hbm.at[pl.ds(0, CHUNK)],
                              abuf.at[b], isems[b]).wait()

    def _start_gather(b):
        pltpu.async_copy(feat_hbm.at[cbuf.at[b]], rows.at[b], gsems[b])

    def _wait_gather(b):
        pltpu.make_async_copy(
            feat_hbm.at[pl.ds(0, CHUNK)], rows.at[b], gsems[b]).wait()

    def _scale_buf(b):
        def _scale(g, carry2):
            av16 = abuf[b, pl.ds(g * 16, 16)]
            for k in range(16):
                av = jnp.full((16,), av16[k], jnp.float32)
                r = g * 16 + k
                for j in range(D_PAD // 16):
                    sl = pl.ds(j * 16, 16)
                    rows[b, r, sl] = rows[b, r, sl] * av
            return carry2
        lax.fori_loop(0, CHUNK // 16, _scale, 0)

    def _start_scat(b):
        pltpu.async_copy(rows.at[b], acc.at[sbuf.at[b]], ssems[b], add=True)

    def _wait_scat(b):
        pltpu.make_async_copy(
            feat_hbm.at[pl.ds(0, CHUNK)], rows.at[b], ssems[b]).wait()

    def _iter(ci, b, first=False, last=False):
        o = 1 - b
        _wait_gather(b)
        _scale_buf(b)
        if not first:
            _wait_scat(o)
        for q in range(CHUNK // 16):
            sl = pl.ds(q * 16, 16)
            sbuf[b, sl] = rbuf[b, sl]
        _start_scat(b)
        if not last:
            c2 = jnp.minimum(ci + 2, NCHUNK - 1)
            _start_idx(c2, b)
            _wait_idx(o)
            _start_gather(o)

    def _zrow(i, carry):
        for j in range(D_PAD // 16):
            rows[0, i, pl.ds(j * 16, 16)] = jnp.zeros((16,), jnp.float32)
        return carry
    lax.fori_loop(0, CHUNK, _zrow, 0)
    _start_idx(0, 0)
    r0base = s * ROWS_PER_TILE
    nfull = ROWS_PER_TILE // CHUNK
    for k in range(nfull):
        pltpu.sync_copy(rows.at[0],
                        acc.at[pl.ds(r0base + k * CHUNK, CHUNK)])
    rem = ROWS_PER_TILE - nfull * CHUNK
    if rem:
        pltpu.sync_copy(rows.at[0, pl.ds(0, rem)],
                        acc.at[pl.ds(r0base + nfull * CHUNK, rem)])
    plsc.subcore_barrier()

    _wait_idx(0)
    _start_gather(0)
    _start_idx(1, 1)
    _iter(0, 0, first=True)

    def _pair(i, carry):
        c0 = 2 * i + 1
        _iter(c0, 1)
        _iter(c0 + 1, 0)
        return carry
    lax.fori_loop(0, (NCHUNK - 2) // 2, _pair, 0)
    _iter(NCHUNK - 1, 1, last=True)
    _wait_scat(1)
    _wait_idx(0)

    toff = pl.multiple_of(ebase + NCHUNK * CHUNK, 8)
    pltpu.async_copy(eidx_hbm.at[0, pl.ds(toff, TAIL)], rt.at[0], semi0)
    pltpu.async_copy(eidx_hbm.at[1, pl.ds(toff, TAIL)], ct.at[0], semi0)
    pltpu.async_copy(adj_hbm.at[pl.ds(toff, TAIL)], at_.at[0], semi0)
    tsrc = eidx_hbm.at[0, pl.ds(0, TAIL)]
    pltpu.make_async_copy(tsrc, rt.at[0], semi0).wait()
    pltpu.make_async_copy(tsrc, ct.at[0], semi0).wait()
    pltpu.make_async_copy(adj_hbm.at[pl.ds(0, TAIL)], at_.at[0], semi0).wait()
    pltpu.async_copy(feat_hbm.at[ct.at[0]], rowt, semg0).wait()
    av16 = at_[0, pl.ds(0, 16)]
    for k in range(TAIL):
        av = jnp.full((16,), av16[k], jnp.float32)
        for j in range(D_PAD // 16):
            sl = pl.ds(j * 16, 16)
            rowt[k, sl] = rowt[k, sl] * av
    pltpu.sync_copy(rowt, acc.at[rt.at[0]], add=True)

    plsc.subcore_barrier()
    nout = ROWS_PER_TILE // CHUNK
    for k in range(nout):
        r0 = r0base + k * CHUNK
        pltpu.sync_copy(acc.at[pl.ds(r0, CHUNK)],
                        out_hbm.at[c, pl.ds(r0, CHUNK)])
    if rem:
        r0 = r0base + nout * CHUNK
        pltpu.sync_copy(acc.at[pl.ds(r0, rem)],
                        out_hbm.at[c, pl.ds(r0, rem)])


@functools.cache
def _make_spmm():
    return pl.kernel(
        _spmm_body,
        out_type=jax.ShapeDtypeStruct((NC, N_NODES, D_PAD), jnp.float32),
        mesh=plsc.VectorSubcoreMesh(
            core_axis_name="c", subcore_axis_name="s",
            num_cores=NC, num_subcores=NS),
        scratch_types=[
            pltpu.VMEM((2, CHUNK), jnp.int32),
            pltpu.VMEM((2, CHUNK), jnp.int32),
            pltpu.VMEM((2, CHUNK), jnp.float32),
            pltpu.VMEM((2, CHUNK), jnp.int32),
            pltpu.VMEM((2, CHUNK, D_PAD), jnp.float32),
            pltpu.VMEM((1, TAIL), jnp.int32),
            pltpu.VMEM((1, TAIL), jnp.int32),
            pltpu.VMEM((1, TAIL), jnp.float32),
            pltpu.VMEM((TAIL, D_PAD), jnp.float32),
            pltpu.VMEM_SHARED((N_NODES, D_PAD), jnp.float32),
            pltpu.SemaphoreType.DMA,
            pltpu.SemaphoreType.DMA,
            pltpu.SemaphoreType.DMA,
            pltpu.SemaphoreType.DMA,
            pltpu.SemaphoreType.DMA,
            pltpu.SemaphoreType.DMA,
        ],
        compiler_params=pltpu.CompilerParams(
            use_tc_tiling_on_sc=False, needs_layout_passes=False),
    )


def _stage3_body(p_ref, q_ref, b_ref, out_ref):
    p = p_ref[0] + p_ref[1]
    denom = jnp.dot(p, q_ref[...], preferred_element_type=jnp.float32)
    out_ref[...] = p[:, :D_FLAT] / (denom + 1e-9) + b_ref[...]


def _stage3(partials, biasf):
    blk = 1000
    grid = N_NODES // blk
    return pl.pallas_call(
        _stage3_body,
        grid=(grid,),
        in_specs=[
            pl.BlockSpec((2, blk, D_PAD), lambda i: (0, i, 0)),
            pl.BlockSpec((D_PAD, D_FLAT), lambda i: (0, 0)),
            pl.BlockSpec((1, D_FLAT), lambda i: (0, 0)),
        ],
        out_specs=pl.BlockSpec((blk, D_FLAT), lambda i: (i, 0)),
        out_shape=jax.ShapeDtypeStruct((N_NODES, D_FLAT), jnp.float32),
    )(partials, _Q, biasf)


def kernel(x, edge_index, adj_values, W, a2, bias):
    feat = _stage1(x, W, a2.reshape(1, D_FLAT))
    eidx = edge_index.astype(jnp.int32)
    adj = adj_values
    partials = _make_spmm()(eidx, adj, feat)
    return _stage3(partials, bias.reshape(1, D_FLAT))


# speedup vs baseline: 1.0733x; 1.0733x over previous
import functools

import jax
import jax.numpy as jnp
import numpy as np
from jax import lax
from jax.experimental import pallas as pl
from jax.experimental.pallas import tpu as pltpu
from jax.experimental.pallas import tpu_sc as plsc

N_NODES = 10000
N_EDGES = 320000
D_IN = 128
D_OUT = 16
N_HEAD = 8
D_FLAT = D_OUT * N_HEAD
D_MSG = (D_OUT + 1) * N_HEAD
D_PAD = 144

NC, NS = 2, 16
NW = NC * NS
EPW = N_EDGES // NW
CHUNK = 128
NCHUNK = EPW // CHUNK
TAIL = EPW - NCHUNK * CHUNK
ROWS_PER_TILE = N_NODES // NS

_P = np.tile(np.eye(N_HEAD, dtype=np.float32), (D_OUT, D_OUT))

_Q_np = np.zeros((D_PAD, D_FLAT), dtype=np.float32)
for _j in range(N_HEAD):
    for _i in range(D_OUT):
        _Q_np[D_FLAT + _j, _i * N_HEAD + _j] = 1.0
_Q = _Q_np


def _stage1_body(x_ref, w_ref, a2_ref, p_ref, out_ref):
    s0 = jnp.dot(x_ref[...], w_ref[...], preferred_element_type=jnp.float32)
    t = s0 * a2_ref[...]
    attn2b = jnp.dot(t, p_ref[...], preferred_element_type=jnp.float32)
    z = attn2b + jnp.sqrt(attn2b * attn2b + 1.0)
    out_ref[...] = jnp.concatenate(
        [s0 * z, z[:, :N_HEAD], jnp.zeros_like(z[:, :N_HEAD])], axis=1)


def _stage1(x, W, a2f):
    blk = 1000
    grid = N_NODES // blk
    return pl.pallas_call(
        _stage1_body,
        grid=(grid,),
        in_specs=[
            pl.BlockSpec((blk, D_IN), lambda i: (i, 0)),
            pl.BlockSpec((D_IN, D_FLAT), lambda i: (0, 0)),
            pl.BlockSpec((1, D_FLAT), lambda i: (0, 0)),
            pl.BlockSpec((D_FLAT, D_FLAT), lambda i: (0, 0)),
        ],
        out_specs=pl.BlockSpec((blk, D_PAD), lambda i: (i, 0)),
        out_shape=jax.ShapeDtypeStruct((N_NODES, D_PAD), jnp.float32),
    )(x, W, a2f, _P)


def _spmm_body(eidx_hbm, adj_hbm, feat_hbm, out_hbm,
               rbuf, cbuf, abuf, rows, rt, ct, at_, rowt, acc,
               semi0, semi1, semg0, semg1):
    c = lax.axis_index("c")
    s = lax.axis_index("s")
    wid = s * NC + c

    isems = (semi0, semi1)
    gsems = (semg0, semg1)

    ebase = wid * EPW

    def _start_idx(ci, b):
        off = pl.multiple_of(ebase + ci * CHUNK, 8)
        pltpu.async_copy(eidx_hbm.at[0, pl.ds(off, CHUNK)],
                         rbuf.at[b], isems[b])
        pltpu.async_copy(eidx_hbm.at[1, pl.ds(off, CHUNK)],
                         cbuf.at[b], isems[b])
        pltpu.async_copy(adj_hbm.at[pl.ds(off, CHUNK)],
                         abuf.at[b], isems[b])

    def _wait_idx(b):
        src = eidx_hbm.at[0, pl.ds(0, CHUNK)]
        pltpu.make_async_copy(src, rbuf.at[b], isems[b]).wait()
        pltpu.make_async_copy(src, cbuf.at[b], isems[b]).wait()
        pltpu.make_async_copy(adj_hbm.at[pl.ds(0, CHUNK)],
                              abuf.at[b], isems[b]).wait()

    def _start_gather(b):
        pltpu.async_copy(feat_hbm.at[cbuf.at[b]], rows.at[b], gsems[b])

    def _wait_gather(b):
        pltpu.make_async_copy(
            feat_hbm.at[pl.ds(0, CHUNK)], rows.at[b], gsems[b]).wait()

    def _scale_buf(b):
        def _scale(g, carry2):
            av16 = abuf[b, pl.ds(g * 16, 16)]
            for k in range(16):
                av = jnp.full((16,), av16[k], jnp.float32)
                r = g * 16 + k
                for j in range(D_PAD // 16):
                    sl = pl.ds(j * 16, 16)
                    rows[b, r, sl] = rows[b, r, sl] * av
            return carry2
        lax.fori_loop(0, CHUNK // 16, _scale, 0)

    def _consume(b):
        _scale_buf(b)
        pltpu.sync_copy(rows.at[b], acc.at[rbuf.at[b]], add=True)

    def _zrow(i, carry):
        for j in range(D_PAD // 16):
            rows[0, i, pl.ds(j * 16, 16)] = jnp.zeros((16,), jnp.float32)
        return carry
    lax.fori_loop(0, CHUNK, _zrow, 0)
    _start_idx(0, 0)
    r0base = s * ROWS_PER_TILE
    nfull = ROWS_PER_TILE // CHUNK
    for k in range(nfull):
        pltpu.sync_copy(rows.at[0],
                        acc.at[pl.ds(r0base + k * CHUNK, CHUNK)])
    rem = ROWS_PER_TILE - nfull * CHUNK
    if rem:
        pltpu.sync_copy(rows.at[0, pl.ds(0, rem)],
                        acc.at[pl.ds(r0base + nfull * CHUNK, rem)])
    plsc.subcore_barrier()

    _wait_idx(0)
    _start_gather(0)
    _start_idx(1, 1)

    def _pair(i, carry):
        c0 = 2 * i
        c2 = jnp.minimum(c0 + 2, NCHUNK - 1)
        c3 = jnp.minimum(c0 + 3, NCHUNK - 1)
        _wait_gather(0)
        _wait_idx(1)
        _start_gather(1)
        _consume(0)
        _start_idx(c2, 0)
        _wait_idx(0)
        _start_gather(0)
        _wait_gather(1)
        _consume(1)
        _start_idx(c3, 1)
        return carry
    lax.fori_loop(0, NCHUNK // 2, _pair, 0)
    _wait_gather(0)
    _wait_idx(1)

    toff = pl.multiple_of(ebase + NCHUNK * CHUNK, 8)
    pltpu.async_copy(eidx_hbm.at[0, pl.ds(toff, TAIL)], rt.at[0], semi0)
    pltpu.async_copy(eidx_hbm.at[1, pl.ds(toff, TAIL)], ct.at[0], semi0)
    pltpu.async_copy(adj_hbm.at[pl.ds(toff, TAIL)], at_.at[0], semi0)
    tsrc = eidx_hbm.at[0, pl.ds(0, TAIL)]
    pltpu.make_async_copy(tsrc, rt.at[0], semi0).wait()
    pltpu.make_async_copy(tsrc, ct.at[0], semi0).wait()
    pltpu.make_async_copy(adj_hbm.at[pl.ds(0, TAIL)], at_.at[0], semi0).wait()
    pltpu.async_copy(feat_hbm.at[ct.at[0]], rowt, semg0).wait()
    av16 = at_[0, pl.ds(0, 16)]
    for k in range(TAIL):
        av = jnp.full((16,), av16[k], jnp.float32)
        for j in range(D_PAD // 16):
            sl = pl.ds(j * 16, 16)
            rowt[k, sl] = rowt[k, sl] * av
    pltpu.sync_copy(rowt, acc.at[rt.at[0]], add=True)

    plsc.subcore_barrier()
    nout = ROWS_PER_TILE // CHUNK
    for k in range(nout):
        r0 = r0base + k * CHUNK
        pltpu.sync_copy(acc.at[pl.ds(r0, CHUNK)],
                        out_hbm.at[c, pl.ds(r0, CHUNK)])
    if rem:
        r0 = r0base + nout * CHUNK
        pltpu.sync_copy(acc.at[pl.ds(r0, rem)],
                        out_hbm.at[c, pl.ds(r0, rem)])


@functools.cache
def _make_spmm():
    return pl.kernel(
        _spmm_body,
        out_type=jax.ShapeDtypeStruct((NC, N_NODES, D_PAD), jnp.float32),
        mesh=plsc.VectorSubcoreMesh(
            core_axis_name="c", subcore_axis_name="s",
            num_cores=NC, num_subcores=NS),
        scratch_types=[
            pltpu.VMEM((2, CHUNK), jnp.int32),
            pltpu.VMEM((2, CHUNK), jnp.int32),
            pltpu.VMEM((2, CHUNK), jnp.float32),
            pltpu.VMEM((2, CHUNK, D_PAD), jnp.float32),
            pltpu.VMEM((1, TAIL), jnp.int32),
            pltpu.VMEM((1, TAIL), jnp.int32),
            pltpu.VMEM((1, TAIL), jnp.float32),
            pltpu.VMEM((TAIL, D_PAD), jnp.float32),
            pltpu.VMEM_SHARED((N_NODES, D_PAD), jnp.float32),
            pltpu.SemaphoreType.DMA,
            pltpu.SemaphoreType.DMA,
            pltpu.SemaphoreType.DMA,
            pltpu.SemaphoreType.DMA,
        ],
        compiler_params=pltpu.CompilerParams(
            use_tc_tiling_on_sc=False, needs_layout_passes=False),
    )


def _stage3_body(p_ref, q_ref, b_ref, out_ref):
    p = p_ref[0] + p_ref[1]
    denom = jnp.dot(p, q_ref[...], preferred_element_type=jnp.float32)
    out_ref[...] = p[:, :D_FLAT] / (denom + 1e-9) + b_ref[...]


def _stage3(partials, biasf):
    blk = 1000
    grid = N_NODES // blk
    return pl.pallas_call(
        _stage3_body,
        grid=(grid,),
        in_specs=[
            pl.BlockSpec((2, blk, D_PAD), lambda i: (0, i, 0)),
            pl.BlockSpec((D_PAD, D_FLAT), lambda i: (0, 0)),
            pl.BlockSpec((1, D_FLAT), lambda i: (0, 0)),
        ],
        out_specs=pl.BlockSpec((blk, D_FLAT), lambda i: (i, 0)),
        out_shape=jax.ShapeDtypeStruct((N_NODES, D_FLAT), jnp.float32),
    )(partials, _Q, biasf)


def kernel(x, edge_index, adj_values, W, a2, bias):
    feat = _stage1(x, W, a2.reshape(1, D_FLAT))
    eidx = edge_index.astype(jnp.int32)
    adj = adj_values
    partials = _make_spmm()(eidx, adj, feat)
    return _stage3(partials, bias.reshape(1, D_FLAT))


# speedup vs baseline: 1.1765x; 1.0961x over previous
import functools

import jax
import jax.numpy as jnp
import numpy as np
from jax import lax
from jax.experimental import pallas as pl
from jax.experimental.pallas import tpu as pltpu
from jax.experimental.pallas import tpu_sc as plsc

N_NODES = 10000
N_EDGES = 320000
D_IN = 128
D_OUT = 16
N_HEAD = 8
D_FLAT = D_OUT * N_HEAD
D_MSG = (D_OUT + 1) * N_HEAD
D_PAD = 144

NC, NS = 2, 16
NW = NC * NS
EPW = N_EDGES // NW
CHUNK = 80
NCHUNK = EPW // CHUNK
NBUF = 3
ROWS_PER_TILE = N_NODES // NS

_P = np.tile(np.eye(N_HEAD, dtype=np.float32), (D_OUT, D_OUT))

_Q_np = np.zeros((D_PAD, D_FLAT), dtype=np.float32)
for _j in range(N_HEAD):
    for _i in range(D_OUT):
        _Q_np[D_FLAT + _j, _i * N_HEAD + _j] = 1.0
_Q = _Q_np


def _stage1_body(x_ref, w_ref, a2_ref, p_ref, out_ref):
    s0 = jnp.dot(x_ref[...], w_ref[...], preferred_element_type=jnp.float32)
    t = s0 * a2_ref[...]
    attn2b = jnp.dot(t, p_ref[...], preferred_element_type=jnp.float32)
    z = attn2b + jnp.sqrt(attn2b * attn2b + 1.0)
    out_ref[...] = jnp.concatenate(
        [s0 * z, z[:, :N_HEAD], jnp.zeros_like(z[:, :N_HEAD])], axis=1)


def _stage1(x, W, a2f):
    blk = 1000
    grid = N_NODES // blk
    return pl.pallas_call(
        _stage1_body,
        grid=(grid,),
        in_specs=[
            pl.BlockSpec((blk, D_IN), lambda i: (i, 0)),
            pl.BlockSpec((D_IN, D_FLAT), lambda i: (0, 0)),
            pl.BlockSpec((1, D_FLAT), lambda i: (0, 0)),
            pl.BlockSpec((D_FLAT, D_FLAT), lambda i: (0, 0)),
        ],
        out_specs=pl.BlockSpec((blk, D_PAD), lambda i: (i, 0)),
        out_shape=jax.ShapeDtypeStruct((N_NODES, D_PAD), jnp.float32),
    )(x, W, a2f, _P)


def _spmm_body(eidx_hbm, adj_hbm, feat_hbm, out_hbm,
               rbuf, cbuf, abuf, rows, acc,
               semi0, semi1, semi2, semg0, semg1, semg2):
    c = lax.axis_index("c")
    s = lax.axis_index("s")
    wid = s * NC + c

    isems = (semi0, semi1, semi2)
    gsems = (semg0, semg1, semg2)

    ebase = wid * EPW

    def _start_idx(ci, b):
        off = pl.multiple_of(ebase + ci * CHUNK, 8)
        pltpu.async_copy(eidx_hbm.at[0, pl.ds(off, CHUNK)],
                         rbuf.at[b], isems[b])
        pltpu.async_copy(eidx_hbm.at[1, pl.ds(off, CHUNK)],
                         cbuf.at[b], isems[b])
        pltpu.async_copy(adj_hbm.at[pl.ds(off, CHUNK)],
                         abuf.at[b], isems[b])

    def _wait_idx(b):
        src = eidx_hbm.at[0, pl.ds(0, CHUNK)]
        pltpu.make_async_copy(src, rbuf.at[b], isems[b]).wait()
        pltpu.make_async_copy(src, cbuf.at[b], isems[b]).wait()
        pltpu.make_async_copy(adj_hbm.at[pl.ds(0, CHUNK)],
                              abuf.at[b], isems[b]).wait()

    def _start_gather(b):
        pltpu.async_copy(feat_hbm.at[cbuf.at[b]], rows.at[b], gsems[b])

    def _wait_gather(b):
        pltpu.make_async_copy(
            feat_hbm.at[pl.ds(0, CHUNK)], rows.at[b], gsems[b]).wait()

    def _scale_buf(b):
        def _scale(g, carry2):
            av16 = abuf[b, pl.ds(g * 16, 16)]
            for k in range(16):
                av = jnp.full((16,), av16[k], jnp.float32)
                r = g * 16 + k
                for j in range(D_PAD // 16):
                    sl = pl.ds(j * 16, 16)
                    rows[b, r, sl] = rows[b, r, sl] * av
            return carry2
        lax.fori_loop(0, CHUNK // 16, _scale, 0)

    def _consume(b):
        _scale_buf(b)
        pltpu.sync_copy(rows.at[b], acc.at[rbuf.at[b]], add=True)

    def _zrow(i, carry):
        for j in range(D_PAD // 16):
            rows[0, i, pl.ds(j * 16, 16)] = jnp.zeros((16,), jnp.float32)
        return carry
    lax.fori_loop(0, CHUNK, _zrow, 0)
    _start_idx(0, 0)
    r0base = s * ROWS_PER_TILE
    nfull = ROWS_PER_TILE // CHUNK
    for k in range(nfull):
        pltpu.sync_copy(rows.at[0],
                        acc.at[pl.ds(r0base + k * CHUNK, CHUNK)])
    rem = ROWS_PER_TILE - nfull * CHUNK
    if rem:
        pltpu.sync_copy(rows.at[0, pl.ds(0, rem)],
                        acc.at[pl.ds(r0base + nfull * CHUNK, rem)])
    plsc.subcore_barrier()

    _wait_idx(0)
    _start_gather(0)
    _start_idx(1, 1)
    _wait_idx(1)
    _start_gather(1)
    _start_idx(2, 2)

    def _step(ci, t):
        t2 = (t + 2) % NBUF
        _wait_gather(t)
        _consume(t)
        _start_idx(jnp.minimum(ci + NBUF, NCHUNK - 1), t)
        _wait_idx(t2)
        _start_gather(t2)

    def _triple(i, carry):
        c0 = NBUF * i
        for q in range(NBUF):
            _step(c0 + q, q)
        return carry
    lax.fori_loop(0, (NCHUNK - 2) // NBUF, _triple, 0)
    _step(NCHUNK - 2, (NCHUNK - 2) % NBUF)
    _step(NCHUNK - 1, (NCHUNK - 1) % NBUF)
    _wait_idx(1)
    _wait_gather(0)
    _wait_gather(2)

    plsc.subcore_barrier()
    nout = ROWS_PER_TILE // CHUNK
    for k in range(nout):
        r0 = r0base + k * CHUNK
        pltpu.sync_copy(acc.at[pl.ds(r0, CHUNK)],
                        out_hbm.at[c, pl.ds(r0, CHUNK)])
    if rem:
        r0 = r0base + nout * CHUNK
        pltpu.sync_copy(acc.at[pl.ds(r0, rem)],
                        out_hbm.at[c, pl.ds(r0, rem)])


@functools.cache
def _make_spmm():
    return pl.kernel(
        _spmm_body,
        out_type=jax.ShapeDtypeStruct((NC, N_NODES, D_PAD), jnp.float32),
        mesh=plsc.VectorSubcoreMesh(
            core_axis_name="c", subcore_axis_name="s",
            num_cores=NC, num_subcores=NS),
        scratch_types=[
            pltpu.VMEM((NBUF, CHUNK), jnp.int32),
            pltpu.VMEM((NBUF, CHUNK), jnp.int32),
            pltpu.VMEM((NBUF, CHUNK), jnp.float32),
            pltpu.VMEM((NBUF, CHUNK, D_PAD), jnp.float32),
            pltpu.VMEM_SHARED((N_NODES, D_PAD), jnp.float32),
            pltpu.SemaphoreType.DMA,
            pltpu.SemaphoreType.DMA,
            pltpu.SemaphoreType.DMA,
            pltpu.SemaphoreType.DMA,
            pltpu.SemaphoreType.DMA,
            pltpu.SemaphoreType.DMA,
        ],
        compiler_params=pltpu.CompilerParams(
            use_tc_tiling_on_sc=False, needs_layout_passes=False),
    )


def _stage3_body(p_ref, q_ref, b_ref, out_ref):
    p = p_ref[0] + p_ref[1]
    denom = jnp.dot(p, q_ref[...], preferred_element_type=jnp.float32)
    out_ref[...] = p[:, :D_FLAT] / (denom + 1e-9) + b_ref[...]


def _stage3(partials, biasf):
    blk = 1000
    grid = N_NODES // blk
    return pl.pallas_call(
        _stage3_body,
        grid=(grid,),
        in_specs=[
            pl.BlockSpec((2, blk, D_PAD), lambda i: (0, i, 0)),
            pl.BlockSpec((D_PAD, D_FLAT), lambda i: (0, 0)),
            pl.BlockSpec((1, D_FLAT), lambda i: (0, 0)),
        ],
        out_specs=pl.BlockSpec((blk, D_FLAT), lambda i: (i, 0)),
        out_shape=jax.ShapeDtypeStruct((N_NODES, D_FLAT), jnp.float32),
    )(partials, _Q, biasf)


def kernel(x, edge_index, adj_values, W, a2, bias):
    feat = _stage1(x, W, a2.reshape(1, D_FLAT))
    eidx = edge_index.astype(jnp.int32)
    adj = adj_values
    partials = _make_spmm()(eidx, adj, feat)
    return _stage3(partials, bias.reshape(1, D_FLAT))


# speedup vs baseline: 1.3389x; 1.1381x over previous
import functools

import jax
import jax.numpy as jnp
import numpy as np
from jax import lax
from jax.experimental import pallas as pl
from jax.experimental.pallas import tpu as pltpu
from jax.experimental.pallas import tpu_sc as plsc

N_NODES = 10000
N_EDGES = 320000
D_IN = 128
D_OUT = 16
N_HEAD = 8
D_FLAT = D_OUT * N_HEAD
D_MSG = (D_OUT + 1) * N_HEAD
D_PAD = 144

NC, NS = 2, 16
NW = NC * NS
EPW = N_EDGES // NW
CHUNK = 80
NCHUNK = EPW // CHUNK
NBUF = 3
ROWS_PER_TILE = N_NODES // NS

_P = np.tile(np.eye(N_HEAD, dtype=np.float32), (D_OUT, D_OUT))

_Q_np = np.zeros((D_PAD, D_FLAT), dtype=np.float32)
for _j in range(N_HEAD):
    for _i in range(D_OUT):
        _Q_np[D_FLAT + _j, _i * N_HEAD + _j] = 1.0
_Q = _Q_np


def _stage1_body(x_ref, w_ref, a2_ref, p_ref, out_ref):
    s0 = jnp.dot(x_ref[...], w_ref[...], preferred_element_type=jnp.float32)
    t = s0 * a2_ref[...]
    attn2b = jnp.dot(t, p_ref[...], preferred_element_type=jnp.float32)
    z = attn2b + jnp.sqrt(attn2b * attn2b + 1.0)
    out_ref[...] = jnp.concatenate(
        [s0 * z, z[:, :N_HEAD], jnp.zeros_like(z[:, :N_HEAD])], axis=1)


def _stage1(x, W, a2f):
    blk = 1000
    grid = N_NODES // blk
    return pl.pallas_call(
        _stage1_body,
        grid=(grid,),
        in_specs=[
            pl.BlockSpec((blk, D_IN), lambda i: (i, 0)),
            pl.BlockSpec((D_IN, D_FLAT), lambda i: (0, 0)),
            pl.BlockSpec((1, D_FLAT), lambda i: (0, 0)),
            pl.BlockSpec((D_FLAT, D_FLAT), lambda i: (0, 0)),
        ],
        out_specs=pl.BlockSpec((blk, D_PAD), lambda i: (i, 0)),
        out_shape=jax.ShapeDtypeStruct((N_NODES, D_PAD), jnp.float32),
    )(x, W, a2f, _P)


def _spmm_body(eidx_hbm, adj_hbm, feat_hbm, out_hbm,
               ibuf, abuf, rows, acc,
               semi0, semi1, semi2, semg0, semg1, semg2):
    c = lax.axis_index("c")
    s = lax.axis_index("s")
    wid = s * NC + c

    isems = (semi0, semi1, semi2)
    gsems = (semg0, semg1, semg2)

    ebase = wid * EPW

    def _start_idx(ci, b):
        off = pl.multiple_of(ebase + ci * CHUNK, 8)
        pltpu.async_copy(eidx_hbm.at[:, pl.ds(off, CHUNK)],
                         ibuf.at[b], isems[b])
        pltpu.async_copy(adj_hbm.at[pl.ds(off, CHUNK)],
                         abuf.at[b], isems[b])

    def _wait_idx(b):
        pltpu.make_async_copy(eidx_hbm.at[:, pl.ds(0, CHUNK)],
                              ibuf.at[b], isems[b]).wait()
        pltpu.make_async_copy(adj_hbm.at[pl.ds(0, CHUNK)],
                              abuf.at[b], isems[b]).wait()

    def _start_gather(b):
        pltpu.async_copy(feat_hbm.at[ibuf.at[b, 1]], rows.at[b], gsems[b])

    def _wait_gather(b):
        pltpu.make_async_copy(
            feat_hbm.at[pl.ds(0, CHUNK)], rows.at[b], gsems[b]).wait()

    def _scale_buf(b):
        @functools.partial(plsc.parallel_loop, 0, CHUNK // 16)
        def _scale(g):
            av16 = abuf[b, pl.ds(g * 16, 16)]
            for k in range(16):
                av = jnp.full((16,), av16[k], jnp.float32)
                r = g * 16 + k
                for j in range(D_PAD // 16):
                    sl = pl.ds(j * 16, 16)
                    rows[b, r, sl] = rows[b, r, sl] * av

    def _consume(b):
        _scale_buf(b)
        pltpu.sync_copy(rows.at[b], acc.at[ibuf.at[b, 0]], add=True)

    def _zrow(i, carry):
        for j in range(D_PAD // 16):
            rows[0, i, pl.ds(j * 16, 16)] = jnp.zeros((16,), jnp.float32)
        return carry
    lax.fori_loop(0, CHUNK, _zrow, 0)
    _start_idx(0, 0)
    r0base = s * ROWS_PER_TILE
    nfull = ROWS_PER_TILE // CHUNK
    for k in range(nfull):
        pltpu.sync_copy(rows.at[0],
                        acc.at[pl.ds(r0base + k * CHUNK, CHUNK)])
    rem = ROWS_PER_TILE - nfull * CHUNK
    if rem:
        pltpu.sync_copy(rows.at[0, pl.ds(0, rem)],
                        acc.at[pl.ds(r0base + nfull * CHUNK, rem)])
    plsc.subcore_barrier()

    _wait_idx(0)
    _start_gather(0)
    _start_idx(1, 1)
    _wait_idx(1)
    _start_gather(1)
    _start_idx(2, 2)

    def _step(ci, t):
        t2 = (t + 2) % NBUF
        _wait_gather(t)
        _consume(t)
        _start_idx(jnp.minimum(ci + NBUF, NCHUNK - 1), t)
        _wait_idx(t2)
        _start_gather(t2)

    def _triple(i, carry):
        c0 = NBUF * i
        for q in range(NBUF):
            _step(c0 + q, q)
        return carry
    lax.fori_loop(0, (NCHUNK - 2) // NBUF, _triple, 0)
    _step(NCHUNK - 2, (NCHUNK - 2) % NBUF)
    _step(NCHUNK - 1, (NCHUNK - 1) % NBUF)
    _wait_idx(1)
    _wait_gather(0)
    _wait_gather(2)

    plsc.subcore_barrier()
    nout = ROWS_PER_TILE // CHUNK
    for k in range(nout):
        r0 = r0base + k * CHUNK
        pltpu.sync_copy(acc.at[pl.ds(r0, CHUNK)],
                        out_hbm.at[c, pl.ds(r0, CHUNK)])
    if rem:
        r0 = r0base + nout * CHUNK
        pltpu.sync_copy(acc.at[pl.ds(r0, rem)],
                        out_hbm.at[c, pl.ds(r0, rem)])


@functools.cache
def _make_spmm():
    return pl.kernel(
        _spmm_body,
        out_type=jax.ShapeDtypeStruct((NC, N_NODES, D_PAD), jnp.float32),
        mesh=plsc.VectorSubcoreMesh(
            core_axis_name="c", subcore_axis_name="s",
            num_cores=NC, num_subcores=NS),
        scratch_types=[
            pltpu.VMEM((NBUF, 2, CHUNK), jnp.int32),
            pltpu.VMEM((NBUF, CHUNK), jnp.float32),
            pltpu.VMEM((NBUF, CHUNK, D_PAD), jnp.float32),
            pltpu.VMEM_SHARED((N_NODES, D_PAD), jnp.float32),
            pltpu.SemaphoreType.DMA,
            pltpu.SemaphoreType.DMA,
            pltpu.SemaphoreType.DMA,
            pltpu.SemaphoreType.DMA,
            pltpu.SemaphoreType.DMA,
            pltpu.SemaphoreType.DMA,
        ],
        compiler_params=pltpu.CompilerParams(
            use_tc_tiling_on_sc=False, needs_layout_passes=False),
    )


def _stage3_body(p_ref, q_ref, b_ref, out_ref):
    p = p_ref[0] + p_ref[1]
    denom = jnp.dot(p, q_ref[...], preferred_element_type=jnp.float32)
    out_ref[...] = p[:, :D_FLAT] / (denom + 1e-9) + b_ref[...]


def _stage3(partials, biasf):
    blk = 1000
    grid = N_NODES // blk
    return pl.pallas_call(
        _stage3_body,
        grid=(grid,),
        in_specs=[
            pl.BlockSpec((2, blk, D_PAD), lambda i: (0, i, 0)),
            pl.BlockSpec((D_PAD, D_FLAT), lambda i: (0, 0)),
            pl.BlockSpec((1, D_FLAT), lambda i: (0, 0)),
        ],
        out_specs=pl.BlockSpec((blk, D_FLAT), lambda i: (i, 0)),
        out_shape=jax.ShapeDtypeStruct((N_NODES, D_FLAT), jnp.float32),
    )(partials, _Q, biasf)


def kernel(x, edge_index, adj_values, W, a2, bias):
    feat = _stage1(x, W, a2.reshape(1, D_FLAT))
    eidx = edge_index.astype(jnp.int32)
    adj = adj_values
    partials = _make_spmm()(eidx, adj, feat)
    return _stage3(partials, bias.reshape(1, D_FLAT))
